# baseline (speedup 1.0000x reference)
import functools

import jax
import jax.numpy as jnp
from jax import lax
from jax.experimental import pallas as pl
from jax.experimental.pallas import tpu as pltpu
from jax.experimental.pallas import tpu_sc as plsc

N = 10000
D = 128
E = 320000
NS = 16
CHUNK = 64
CH_PER_TILE = 320
E_PAD = NS * CH_PER_TILE * CHUNK
N_PAD = 10112
ROWS_PER_TILE = N_PAD // NS
NB = 4
SCH = 8
NSI = 4
NSCH = CH_PER_TILE // SCH
UNROLL = NSI * SCH


@functools.partial(
    pl.kernel,
    out_type=jax.ShapeDtypeStruct((N_PAD, D), jnp.float32),
    mesh=plsc.VectorSubcoreMesh(
        core_axis_name="c", subcore_axis_name="s", num_cores=1, num_subcores=NS
    ),
    scratch_types=[
        pltpu.VMEM((NB, CHUNK, D), jnp.float32),
        pltpu.VMEM((NSI, SCH, CHUNK), jnp.int32),
        pltpu.VMEM((NSI, SCH, CHUNK), jnp.int32),
        pltpu.VMEM_SHARED((N_PAD, D), jnp.float32),
        pltpu.SemaphoreType.DMA((NB,)),
        pltpu.SemaphoreType.DMA((NB,)),
        pltpu.SemaphoreType.DMA((NSI,)),
        pltpu.SemaphoreType.DMA((NSI,)),
    ],
)
def _sc_aggregate(x_hbm, src_hbm, dst_hbm, zeros_hbm, out_hbm,
                  rows_v, si_v, di_v, agg_sh, sg, ss, s_si, s_di):
    sid = lax.axis_index("s")
    stripe = sid * ROWS_PER_TILE
    s_last = NSCH - 1

    pltpu.sync_copy(zeros_hbm, agg_sh.at[pl.ds(stripe, ROWS_PER_TILE)])
    plsc.subcore_barrier()

    def i_start(s, islot):
        pltpu.async_copy(src_hbm.at[sid, s], si_v.at[islot], s_si.at[islot])
        pltpu.async_copy(dst_hbm.at[sid, s], di_v.at[islot], s_di.at[islot])

    def i_wait(s, islot):
        pltpu.make_async_copy(src_hbm.at[sid, s], si_v.at[islot],
                              s_si.at[islot]).wait()
        pltpu.make_async_copy(dst_hbm.at[sid, s], di_v.at[islot],
                              s_di.at[islot]).wait()

    def g_start(b, islot, m):
        pltpu.async_copy(x_hbm.at[si_v.at[islot, m]], rows_v.at[b], sg.at[b])

    def g_wait(b, islot, m):
        pltpu.make_async_copy(x_hbm.at[si_v.at[islot, m]], rows_v.at[b],
                              sg.at[b]).wait()

    def s_start(b, islot, m):
        pltpu.async_copy(rows_v.at[b], agg_sh.at[di_v.at[islot, m]],
                         ss.at[b], add=True)

    def s_wait(b, islot, m):
        pltpu.make_async_copy(rows_v.at[b], agg_sh.at[di_v.at[islot, m]],
                              ss.at[b]).wait()

    def step(s0, t, skip):
        b, bm = t % NB, (t - 1) % NB
        sl, slm = (t // SCH) % NSI, ((t - 1) // SCH) % NSI
        m, mm = t % SCH, (t - 1) % SCH
        if t % SCH == 0:
            i_wait(s0 + t // SCH, sl)
            i_start(jnp.minimum(s0 + t // SCH + 2, s_last), (sl + 2) % NSI)
        if skip <= -4:
            s_wait(b, (sl - (1 if m < 4 else 0)) % NSI, (m - 4) % SCH)
        g_start(b, sl, m)
        if skip <= -1:
            g_wait(bm, slm, mm)
            s_start(bm, slm, mm)

    i_start(0, 0)
    i_start(1, 1)
    for t in range(UNROLL):
        step(0, t, -t)

    def body(jj, carry):
        for t in range(UNROLL):
            step(jj * NSI, t, -4)
        return carry

    lax.fori_loop(1, NSCH // NSI, body, 0)

    g_wait(3, (NSCH - 1) % NSI, SCH - 1)
    s_start(3, (NSCH - 1) % NSI, SCH - 1)
    for c in range(CH_PER_TILE - 4, CH_PER_TILE):
        s_wait(c % NB, (c // SCH) % NSI, c % SCH)
    i_wait(s_last, 0)
    i_wait(s_last, 1)
    plsc.subcore_barrier()

    pltpu.sync_copy(agg_sh.at[pl.ds(stripe, ROWS_PER_TILE)],
                    out_hbm.at[pl.ds(stripe, ROWS_PER_TILE)])


def _tc_body(p_ref, x_ref, wl_ref, wr_ref, b_ref, o_ref, *, relu):
    acc = jnp.dot(p_ref[...], wl_ref[...], preferred_element_type=jnp.float32)
    acc += jnp.dot(x_ref[...], wr_ref[...], preferred_element_type=jnp.float32)
    acc += b_ref[...]
    o_ref[...] = jnp.maximum(acc, 0.0) if relu else acc


def _tc_combine(p, x, wlT, wrT, b, relu):
    blk = 2000
    grid = (N // blk,)
    row_spec = pl.BlockSpec((blk, D), lambda i: (i, 0))
    full_spec = pl.BlockSpec((D, D), lambda i: (0, 0))
    bias_spec = pl.BlockSpec((1, D), lambda i: (0, 0))
    return pl.pallas_call(
        functools.partial(_tc_body, relu=relu),
        grid=grid,
        in_specs=[row_spec, row_spec, full_spec, full_spec, bias_spec],
        out_specs=row_spec,
        out_shape=jax.ShapeDtypeStruct((N, D), jnp.float32),
    )(p, x, wlT, wrT, b.reshape(1, D))


def kernel(x, edge_index, Wl1, bl1, Wr1, Wl2, bl2, Wr2):
    src = edge_index[0]
    dst = edge_index[1]
    pad = E_PAD - E
    src_p = jnp.concatenate([src, jnp.zeros((pad,), jnp.int32)])
    dst_p = jnp.concatenate([dst, jnp.full((pad,), N, jnp.int32)])
    src_p = src_p.reshape(NS, NSCH, SCH, CHUNK)
    dst_p = dst_p.reshape(NS, NSCH, SCH, CHUNK)
    zeros = jnp.zeros((ROWS_PER_TILE, D), jnp.float32)

    p1 = _sc_aggregate(x, src_p, dst_p, zeros)
    h = _tc_combine(p1[:N], x, Wl1.T, Wr1.T, bl1, relu=True)
    p2 = _sc_aggregate(h, src_p, dst_p, zeros)
    return _tc_combine(p2[:N], h, Wl2.T, Wr2.T, bl2, relu=False)


# speedup vs baseline: 1.0780x; 1.0780x over previous
import functools

import jax
import jax.numpy as jnp
from jax import lax
from jax.experimental import pallas as pl
from jax.experimental.pallas import tpu as pltpu
from jax.experimental.pallas import tpu_sc as plsc

N = 10000
D = 128
E = 320000
NC = 2
NS = 16
CHUNK = 64
CH0 = 248
CH1 = 72
E_PAD = NS * (CH0 + CH1) * CHUNK
N_PAD = 10112
ROWS_PER_TILE = N_PAD // NS
NB = 4
NI = 8
UNROLL = 8


def _run_slab(x_hbm, src_hbm, dst_hbm, src_v, rows_v, di_v, agg_sh,
              sg, ss, sd, sid, nch):
    last = nch - 1
    pltpu.sync_copy(src_hbm.at[sid], src_v.at[pl.ds(0, nch * CHUNK)])

    def di_start(c, ib):
        pltpu.async_copy(dst_hbm.at[sid, c], di_v.at[ib], sd.at[ib])

    def di_wait(c, ib):
        pltpu.make_async_copy(dst_hbm.at[sid, c], di_v.at[ib], sd.at[ib]).wait()

    def g_start(c, b):
        pltpu.async_copy(x_hbm.at[src_v.at[pl.ds(c * CHUNK, CHUNK)]],
                         rows_v.at[b], sg.at[b])

    def g_wait(c, b):
        pltpu.make_async_copy(x_hbm.at[src_v.at[pl.ds(c * CHUNK, CHUNK)]],
                              rows_v.at[b], sg.at[b]).wait()

    def s_start(b, ib):
        pltpu.async_copy(rows_v.at[b], agg_sh.at[di_v.at[ib]], ss.at[b],
                         add=True)

    def s_wait(b, ib):
        pltpu.make_async_copy(rows_v.at[b], agg_sh.at[di_v.at[ib]],
                              ss.at[b]).wait()

    def step(j, k, skip):
        b, bm, ibm, ibn = k % NB, (k - 1) % NB, (k - 1) % NI, (k + 2) % NI
        if skip <= -4:
            s_wait(b, (k - 4) % NI)
        di_start(jnp.minimum(j + 2, last), ibn)
        g_start(j, b)
        if skip <= -1:
            g_wait(j - 1, bm)
            di_wait(j - 1, ibm)
            s_start(bm, ibm)

    di_start(0, 0)
    di_start(1, 1)
    for j in range(UNROLL):
        step(j, j, -j)

    def body(jj, carry):
        for k in range(UNROLL):
            step(jj * UNROLL + k, k, -4)
        return carry

    lax.fori_loop(1, nch // UNROLL, body, 0)

    g_wait(last, last % NB)
    di_wait(last, last % NI)
    s_start(last % NB, last % NI)
    for c in range(last - 3, last + 1):
        s_wait(c % NB, c % NI)
    di_wait(last, 0)
    di_wait(last, 1)


@functools.partial(
    pl.kernel,
    out_type=jax.ShapeDtypeStruct((NC, N_PAD, D), jnp.float32),
    mesh=plsc.VectorSubcoreMesh(
        core_axis_name="c", subcore_axis_name="s", num_cores=NC, num_subcores=NS
    ),
    scratch_types=[
        pltpu.VMEM((CH0 * CHUNK,), jnp.int32),
        pltpu.VMEM((NB, CHUNK, D), jnp.float32),
        pltpu.VMEM((NI, CHUNK), jnp.int32),
        pltpu.VMEM_SHARED((N_PAD, D), jnp.float32),
        pltpu.SemaphoreType.DMA((NB,)),
        pltpu.SemaphoreType.DMA((NB,)),
        pltpu.SemaphoreType.DMA((NI,)),
    ],
)
def _sc_aggregate(x_hbm, src0_hbm, dst0_hbm, src1_hbm, dst1_hbm,
                  out_hbm, src_v, rows_v, di_v, agg_sh, sg, ss, sd):
    cid = lax.axis_index("c")
    sid = lax.axis_index("s")
    stripe = sid * ROWS_PER_TILE

    def zrow(i, carry):
        rows_v[0, i // 8, pl.ds((i % 8) * 16, 16)] = jnp.zeros(
            (16,), jnp.float32)
        return carry

    lax.fori_loop(0, CHUNK * (D // 16), zrow, 0)
    for q in range(ROWS_PER_TILE // CHUNK):
        pltpu.sync_copy(rows_v.at[0],
                        agg_sh.at[pl.ds(stripe + q * CHUNK, CHUNK)])
    rem = ROWS_PER_TILE % CHUNK
    pltpu.sync_copy(rows_v.at[0, pl.ds(0, rem)],
                    agg_sh.at[pl.ds(stripe + ROWS_PER_TILE - rem, rem)])
    plsc.subcore_barrier()

    @pl.when(cid == 0)
    def _():
        _run_slab(x_hbm, src0_hbm, dst0_hbm, src_v, rows_v, di_v, agg_sh,
                  sg, ss, sd, sid, CH0)

    @pl.when(cid == 1)
    def _():
        _run_slab(x_hbm, src1_hbm, dst1_hbm, src_v, rows_v, di_v, agg_sh,
                  sg, ss, sd, sid, CH1)

    plsc.subcore_barrier()

    pltpu.sync_copy(agg_sh.at[pl.ds(stripe, ROWS_PER_TILE)],
                    out_hbm.at[cid, pl.ds(stripe, ROWS_PER_TILE)])


def _tc_body(p0_ref, p1_ref, x_ref, wl_ref, wr_ref, b_ref, o_ref, *, relu):
    agg = p0_ref[...] + p1_ref[...]
    acc = jnp.dot(agg, wl_ref[...], preferred_element_type=jnp.float32)
    acc += jnp.dot(x_ref[...], wr_ref[...], preferred_element_type=jnp.float32)
    acc += b_ref[...]
    o_ref[...] = jnp.maximum(acc, 0.0) if relu else acc


def _tc_combine(p0, p1, x, wlT, wrT, b, relu):
    blk = 2000
    grid = (N // blk,)
    row_spec = pl.BlockSpec((blk, D), lambda i: (i, 0))
    full_spec = pl.BlockSpec((D, D), lambda i: (0, 0))
    bias_spec = pl.BlockSpec((1, D), lambda i: (0, 0))
    return pl.pallas_call(
        functools.partial(_tc_body, relu=relu),
        grid=grid,
        in_specs=[row_spec, row_spec, row_spec, full_spec, full_spec, bias_spec],
        out_specs=row_spec,
        out_shape=jax.ShapeDtypeStruct((N, D), jnp.float32),
    )(p0, p1, x, wlT, wrT, b.reshape(1, D))


def kernel(x, edge_index, Wl1, bl1, Wr1, Wl2, bl2, Wr2):
    src = edge_index[0]
    dst = edge_index[1]
    pad = E_PAD - E
    src_p = jnp.concatenate([src, jnp.zeros((pad,), jnp.int32)])
    dst_p = jnp.concatenate([dst, jnp.full((pad,), N, jnp.int32)])
    cut = NS * CH0 * CHUNK
    src0 = src_p[:cut].reshape(NS, CH0 * CHUNK)
    src1 = src_p[cut:].reshape(NS, CH1 * CHUNK)
    dst0 = dst_p[:cut].reshape(NS, CH0, CHUNK)
    dst1 = dst_p[cut:].reshape(NS, CH1, CHUNK)

    p1 = _sc_aggregate(x, src0, dst0, src1, dst1)
    h = _tc_combine(p1[0, :N], p1[1, :N], x, Wl1.T, Wr1.T, bl1, relu=True)
    p2 = _sc_aggregate(h, src0, dst0, src1, dst1)
    return _tc_combine(p2[0, :N], p2[1, :N], h, Wl2.T, Wr2.T, bl2, relu=False)


# speedup vs baseline: 1.2041x; 1.1170x over previous
import functools

import jax
import jax.numpy as jnp
from jax import lax
from jax.experimental import pallas as pl
from jax.experimental.pallas import tpu as pltpu
from jax.experimental.pallas import tpu_sc as plsc

N = 10000
D = 128
E = 320000
NC = 2
NS = 16
CHUNK = 64
CH0 = 248
CH1 = 72
E_PAD = NS * (CH0 + CH1) * CHUNK
N_PAD = 10112
ROWS_PER_TILE = N_PAD // NS
NB = 4
NI = 8
UNROLL = 8


def _run_slab(x_hbm, src_hbm, dst_hbm, src_v, rows_v, di_v, agg_sh,
              sg, ss, sd, sid, nch):
    last = nch - 1
    pltpu.sync_copy(src_hbm.at[sid], src_v.at[pl.ds(0, nch * CHUNK)])

    def di_start(c, ib):
        pltpu.async_copy(dst_hbm.at[sid, c], di_v.at[ib], sd.at[ib])

    def di_wait(c, ib):
        pltpu.make_async_copy(dst_hbm.at[sid, c], di_v.at[ib], sd.at[ib]).wait()

    def g_start(c, b):
        pltpu.async_copy(x_hbm.at[src_v.at[pl.ds(c * CHUNK, CHUNK)]],
                         rows_v.at[b], sg.at[b])

    def g_wait(c, b):
        pltpu.make_async_copy(x_hbm.at[src_v.at[pl.ds(c * CHUNK, CHUNK)]],
                              rows_v.at[b], sg.at[b]).wait()

    def s_start(b, ib):
        pltpu.async_copy(rows_v.at[b], agg_sh.at[di_v.at[ib]], ss.at[b],
                         add=True)

    def s_wait(b, ib):
        pltpu.make_async_copy(rows_v.at[b], agg_sh.at[di_v.at[ib]],
                              ss.at[b]).wait()

    def step(j, k, skip):
        b, bm, ibm, ibn = k % NB, (k - 1) % NB, (k - 1) % NI, (k + 2) % NI
        if skip <= -4:
            s_wait(b, (k - 4) % NI)
        di_start(jnp.minimum(j + 2, last), ibn)
        g_start(j, b)
        if skip <= -1:
            g_wait(j - 1, bm)
            di_wait(j - 1, ibm)
            s_start(bm, ibm)

    di_start(0, 0)
    di_start(1, 1)
    for j in range(UNROLL):
        step(j, j, -j)

    def body(jj, carry):
        for k in range(UNROLL):
            step(jj * UNROLL + k, k, -4)
        return carry

    lax.fori_loop(1, nch // UNROLL, body, 0)

    g_wait(last, last % NB)
    di_wait(last, last % NI)
    s_start(last % NB, last % NI)
    for c in range(last - 3, last + 1):
        s_wait(c % NB, c % NI)
    di_wait(last, 0)
    di_wait(last, 1)


@functools.partial(
    pl.kernel,
    out_type=jax.ShapeDtypeStruct((NC, N_PAD, D), jnp.float32),
    mesh=plsc.VectorSubcoreMesh(
        core_axis_name="c", subcore_axis_name="s", num_cores=NC, num_subcores=NS
    ),
    scratch_types=[
        pltpu.VMEM((CH0 * CHUNK,), jnp.int32),
        pltpu.VMEM((NB, CHUNK, D), jnp.float32),
        pltpu.VMEM((NI, CHUNK), jnp.int32),
        pltpu.VMEM_SHARED((N_PAD, D), jnp.float32),
        pltpu.SemaphoreType.DMA((NB,)),
        pltpu.SemaphoreType.DMA((NB,)),
        pltpu.SemaphoreType.DMA((NI,)),
    ],
)
def _sc_aggregate(x_hbm, src0_hbm, dst0_hbm, src1_hbm, dst1_hbm, zeros_hbm,
                  out_hbm, src_v, rows_v, di_v, agg_sh, sg, ss, sd):
    cid = lax.axis_index("c")
    sid = lax.axis_index("s")
    stripe = sid * ROWS_PER_TILE

    pltpu.sync_copy(zeros_hbm, agg_sh.at[pl.ds(stripe, ROWS_PER_TILE)])
    plsc.subcore_barrier()

    @pl.when(cid == 0)
    def _():
        _run_slab(x_hbm, src0_hbm, dst0_hbm, src_v, rows_v, di_v, agg_sh,
                  sg, ss, sd, sid, CH0)

    @pl.when(cid == 1)
    def _():
        _run_slab(x_hbm, src1_hbm, dst1_hbm, src_v, rows_v, di_v, agg_sh,
                  sg, ss, sd, sid, CH1)

    plsc.subcore_barrier()

    pltpu.sync_copy(agg_sh.at[pl.ds(stripe, ROWS_PER_TILE)],
                    out_hbm.at[cid, pl.ds(stripe, ROWS_PER_TILE)])


def _tc_body(p0_ref, p1_ref, x_ref, wl_ref, wr_ref, b_ref, o_ref, *, relu):
    agg = p0_ref[...] + p1_ref[...]
    acc = jnp.dot(agg, wl_ref[...], preferred_element_type=jnp.float32)
    acc += jnp.dot(x_ref[...], wr_ref[...], preferred_element_type=jnp.float32)
    acc += b_ref[...]
    o_ref[...] = jnp.maximum(acc, 0.0) if relu else acc


def _tc_combine(p0, p1, x, wlT, wrT, b, relu):
    blk = 2000
    grid = (N // blk,)
    row_spec = pl.BlockSpec((blk, D), lambda i: (i, 0))
    full_spec = pl.BlockSpec((D, D), lambda i: (0, 0))
    bias_spec = pl.BlockSpec((1, D), lambda i: (0, 0))
    return pl.pallas_call(
        functools.partial(_tc_body, relu=relu),
        grid=grid,
        in_specs=[row_spec, row_spec, row_spec, full_spec, full_spec, bias_spec],
        out_specs=row_spec,
        out_shape=jax.ShapeDtypeStruct((N, D), jnp.float32),
    )(p0, p1, x, wlT, wrT, b.reshape(1, D))


def kernel(x, edge_index, Wl1, bl1, Wr1, Wl2, bl2, Wr2):
    src = edge_index[0]
    dst = edge_index[1]
    pad = E_PAD - E
    src_p = jnp.concatenate([src, jnp.zeros((pad,), jnp.int32)])
    dst_p = jnp.concatenate([dst, jnp.full((pad,), N, jnp.int32)])
    cut = NS * CH0 * CHUNK
    src0 = src_p[:cut].reshape(NS, CH0 * CHUNK)
    src1 = src_p[cut:].reshape(NS, CH1 * CHUNK)
    dst0 = dst_p[:cut].reshape(NS, CH0, CHUNK)
    dst1 = dst_p[cut:].reshape(NS, CH1, CHUNK)
    zeros = jnp.zeros((ROWS_PER_TILE, D), jnp.float32)

    p1 = _sc_aggregate(x, src0, dst0, src1, dst1, zeros)
    h = _tc_combine(p1[0, :N], p1[1, :N], x, Wl1.T, Wr1.T, bl1, relu=True)
    p2 = _sc_aggregate(h, src0, dst0, src1, dst1, zeros)
    return _tc_combine(p2[0, :N], p2[1, :N], h, Wl2.T, Wr2.T, bl2, relu=False)
